# baseline (speedup 1.0000x reference)
import jax
import jax.numpy as jnp
from jax.experimental import pallas as pl
from jax.experimental.pallas import tpu as pltpu


def _linear_kernel(f_ref, w_ref, b_ref, o_ref):
    acc = jax.lax.dot_general(
        f_ref[...], w_ref[...],
        dimension_numbers=(((1,), (1,)), ((), ())),
        preferred_element_type=jnp.float32,
    )
    o_ref[...] = acc + b_ref[...]


def kernel(feat, W, b):
    B, D = feat.shape
    C = W.shape[0]
    Bm = 512
    return pl.pallas_call(
        _linear_kernel,
        grid=(B // Bm,),
        in_specs=[
            pl.BlockSpec((Bm, D), lambda i: (i, 0)),
            pl.BlockSpec((C, D), lambda i: (0, 0)),
            pl.BlockSpec((1, C), lambda i: (0, 0)),
        ],
        out_specs=pl.BlockSpec((Bm, C), lambda i: (i, 0)),
        out_shape=jax.ShapeDtypeStruct((B, C), jnp.float32),
        compiler_params=pltpu.CompilerParams(
            dimension_semantics=("parallel",),
        ),
    )(feat, W, b.reshape(1, C))


# speedup vs baseline: 1.1482x; 1.1482x over previous
import jax
import jax.numpy as jnp
from jax.experimental import pallas as pl
from jax.experimental.pallas import tpu as pltpu


def _linear_kernel(f_ref, w_ref, b_ref, o_ref):
    acc = jax.lax.dot_general(
        f_ref[...], w_ref[...],
        dimension_numbers=(((1,), (1,)), ((), ())),
        preferred_element_type=jnp.float32,
    )
    o_ref[...] = acc + b_ref[...]


def kernel(feat, W, b):
    B, D = feat.shape
    C = W.shape[0]
    Bm = 1024
    return pl.pallas_call(
        _linear_kernel,
        grid=(B // Bm,),
        in_specs=[
            pl.BlockSpec((Bm, D), lambda i: (i, 0)),
            pl.BlockSpec((C, D), lambda i: (0, 0)),
            pl.BlockSpec((1, C), lambda i: (0, 0)),
        ],
        out_specs=pl.BlockSpec((Bm, C), lambda i: (i, 0)),
        out_shape=jax.ShapeDtypeStruct((B, C), jnp.float32),
        compiler_params=pltpu.CompilerParams(
            dimension_semantics=("parallel",),
        ),
    )(feat, W, b.reshape(1, C))
